# Initial kernel scaffold; baseline (speedup 1.0000x reference)
import jax
import jax.numpy as jnp
from jax.experimental import pallas as pl


def kernel(x, edge_index, W, b):
    raise NotImplementedError("write your pallas kernel here")


# speedup vs baseline: 3.4042x; 3.4042x over previous
import functools

import jax
import jax.numpy as jnp
from jax import lax
from jax.experimental import pallas as pl
from jax.experimental.pallas import tpu as pltpu
from jax.experimental.pallas import tpu_sc as plsc

N = 10000
E = 320000
D = 128
CW = 16

NC = 2
NS = 16
NW = NC * NS
EPW = E // NW
K = 80
NCHUNK = EPW // K
NP = 10240
RPS = NP // NS

_MESH = dict(core_axis_name="c", subcore_axis_name="s", num_cores=NC,
             num_subcores=NS)


def _sc_segment_sum(src, dst, x, z2d):
  @functools.partial(
      pl.kernel,
      mesh=plsc.VectorSubcoreMesh(**_MESH),
      out_type=jax.ShapeDtypeStruct((NC * NP, D), jnp.float32),
      scratch_types=[
          pltpu.VMEM_SHARED((NP, D), jnp.float32),
          pltpu.VMEM((K,), jnp.int32),
          pltpu.VMEM((K,), jnp.int32),
          pltpu.VMEM((K, D), jnp.float32),
          pltpu.SemaphoreType.DMA,
      ],
  )
  def body(src_hbm, dst_hbm, x_hbm, z2d_hbm, sum_out, acc_sh,
           src_idx, dst_idx, rows, sem):
    c = lax.axis_index("c")
    s = lax.axis_index("s")
    wid = s * NC + c

    srow0 = pl.multiple_of(s * RPS, 8)
    row0 = pl.multiple_of(c * NP + s * RPS, 8)
    pltpu.sync_copy(z2d_hbm, acc_sh.at[pl.ds(srow0, RPS), :])
    plsc.subcore_barrier()

    base = wid * EPW

    def chunk(j, carry):
      off = base + j * K
      pltpu.sync_copy(src_hbm.at[pl.ds(off, K)], src_idx)
      pltpu.sync_copy(dst_hbm.at[pl.ds(off, K)], dst_idx)
      pltpu.async_copy(x_hbm.at[src_idx], rows, sem).wait()
      pltpu.sync_copy(rows, acc_sh.at[dst_idx], add=True)
      return carry

    lax.fori_loop(0, NCHUNK, chunk, 0)
    plsc.subcore_barrier()

    pltpu.sync_copy(acc_sh.at[pl.ds(srow0, RPS), :],
                    sum_out.at[pl.ds(row0, RPS), :])

  return body(src, dst, x, z2d)


def _sc_degree(dst, zc, ones_h):
  @functools.partial(
      pl.kernel,
      mesh=plsc.VectorSubcoreMesh(**_MESH),
      out_type=jax.ShapeDtypeStruct((NC * NP, CW), jnp.float32),
      scratch_types=[
          pltpu.VMEM_SHARED((NP, CW), jnp.float32),
          pltpu.VMEM((K,), jnp.int32),
          pltpu.VMEM((K, CW), jnp.float32),
      ],
  )
  def body(dst_hbm, zc_hbm, ones_hbm, cnt_out, cnt_sh, dst_idx, ones_v):
    c = lax.axis_index("c")
    s = lax.axis_index("s")
    wid = s * NC + c

    srow0 = pl.multiple_of(s * RPS, 8)
    row0 = pl.multiple_of(c * NP + s * RPS, 8)
    pltpu.sync_copy(zc_hbm, cnt_sh.at[pl.ds(srow0, RPS), :])
    pltpu.sync_copy(ones_hbm, ones_v)
    plsc.subcore_barrier()

    base = wid * EPW

    def chunk(j, carry):
      off = base + j * K
      pltpu.sync_copy(dst_hbm.at[pl.ds(off, K)], dst_idx)
      pltpu.sync_copy(ones_v, cnt_sh.at[dst_idx], add=True)
      return carry

    lax.fori_loop(0, NCHUNK, chunk, 0)
    plsc.subcore_barrier()

    pltpu.sync_copy(cnt_sh.at[pl.ds(srow0, RPS), :],
                    cnt_out.at[pl.ds(row0, RPS), :])

  return body(dst, zc, ones_h)


def _tc_sage(x, sum_parts, cnt_parts, w1t, w2t, b2):
  R = 1000

  def body(x_ref, sp_ref, cp_ref, w1_ref, w2_ref, b_ref, o_ref):
    xb = x_ref[...]
    summed = sp_ref[0] + sp_ref[1]
    cnt = cp_ref[0, :, 0:1] + cp_ref[1, :, 0:1]
    mean = summed / jnp.maximum(cnt, 1.0)
    agg = jnp.where(cnt > 0.0, mean, xb)
    h = (jnp.dot(xb, w1_ref[...], preferred_element_type=jnp.float32)
         + jnp.dot(agg, w2_ref[...], preferred_element_type=jnp.float32)
         + b_ref[...])
    o_ref[...] = jnp.maximum(h, 0.0)

  return pl.pallas_call(
      body,
      grid=(N // R,),
      in_specs=[
          pl.BlockSpec((R, D), lambda i: (i, 0)),
          pl.BlockSpec((NC, R, D), lambda i: (0, i, 0)),
          pl.BlockSpec((NC, R, D), lambda i: (0, i, 0)),
          pl.BlockSpec((D, D), lambda i: (0, 0)),
          pl.BlockSpec((D, D), lambda i: (0, 0)),
          pl.BlockSpec((1, D), lambda i: (0, 0)),
      ],
      out_specs=pl.BlockSpec((R, D), lambda i: (i, 0)),
      out_shape=jax.ShapeDtypeStruct((N, D), jnp.float32),
  )(x, sum_parts, cnt_parts, w1t, w2t, b2)


def kernel(x, edge_index, W, b):
  src = edge_index[0].astype(jnp.int32)
  dst = edge_index[1].astype(jnp.int32)
  z2d = jnp.zeros((RPS, D), jnp.float32)
  zc = jnp.zeros((RPS, CW), jnp.float32)
  ones_h = jnp.ones((K, CW), jnp.float32)
  sum_flat = _sc_segment_sum(src, dst, x, z2d)
  ones_tbl = jnp.ones((N, D), jnp.float32) + sum_flat[:N, :] * 0.0
  cnt_flat = _sc_segment_sum(src, dst, ones_tbl, z2d)
  sum_parts = sum_flat.reshape(NC, NP, D)
  cnt_parts = cnt_flat.reshape(NC, NP, D)
  w1t = W[:, :D].T
  w2t = W[:, D:].T
  b2 = b[None, :]
  return _tc_sage(x, sum_parts, cnt_parts, w1t, w2t, b2)


# speedup vs baseline: 4.7401x; 1.3924x over previous
import functools

import jax
import jax.numpy as jnp
from jax import lax
from jax.experimental import pallas as pl
from jax.experimental.pallas import tpu as pltpu
from jax.experimental.pallas import tpu_sc as plsc

N = 10000
E = 320000
D = 128
CW = 16

NC = 2
NS = 16
NW = NC * NS
EPW = E // NW
K = 80
NCHUNK = EPW // K
NP = 10240
RPS = NP // NS

_MESH = dict(core_axis_name="c", subcore_axis_name="s", num_cores=NC,
             num_subcores=NS)


def _sc_segment_sum(src, dst, x, z2d):
  @functools.partial(
      pl.kernel,
      mesh=plsc.VectorSubcoreMesh(**_MESH),
      out_type=jax.ShapeDtypeStruct((NC * NP, D), jnp.float32),
      scratch_types=[
          pltpu.VMEM_SHARED((NP, D), jnp.float32),
          pltpu.VMEM((K,), jnp.int32),
          pltpu.VMEM((K,), jnp.int32),
          pltpu.VMEM((K, D), jnp.float32),
          pltpu.SemaphoreType.DMA,
      ],
  )
  def body(src_hbm, dst_hbm, x_hbm, z2d_hbm, sum_out, acc_sh,
           src_idx, dst_idx, rows, sem):
    c = lax.axis_index("c")
    s = lax.axis_index("s")
    wid = s * NC + c

    srow0 = pl.multiple_of(s * RPS, 8)
    row0 = pl.multiple_of(c * NP + s * RPS, 8)
    pltpu.sync_copy(z2d_hbm, acc_sh.at[pl.ds(srow0, RPS), :])
    plsc.subcore_barrier()

    base = wid * EPW

    def chunk(j, carry):
      off = base + j * K
      pltpu.sync_copy(src_hbm.at[pl.ds(off, K)], src_idx)
      pltpu.sync_copy(dst_hbm.at[pl.ds(off, K)], dst_idx)
      pltpu.async_copy(x_hbm.at[src_idx], rows, sem).wait()
      pltpu.sync_copy(rows, acc_sh.at[dst_idx], add=True)
      return carry

    lax.fori_loop(0, NCHUNK, chunk, 0)
    plsc.subcore_barrier()

    pltpu.sync_copy(acc_sh.at[pl.ds(srow0, RPS), :],
                    sum_out.at[pl.ds(row0, RPS), :])

  return body(src, dst, x, z2d)


def _sc_degree(dst, zc, ones_h):
  @functools.partial(
      pl.kernel,
      mesh=plsc.VectorSubcoreMesh(**_MESH),
      out_type=jax.ShapeDtypeStruct((NC * NP, D), jnp.float32),
      scratch_types=[
          pltpu.VMEM_SHARED((NP, D), jnp.float32),
          pltpu.VMEM((K,), jnp.int32),
          pltpu.VMEM((K, D), jnp.float32),
      ],
  )
  def body(dst_hbm, zc_hbm, ones_hbm, cnt_out, cnt_sh, dst_idx, ones_v):
    c = lax.axis_index("c")
    s = lax.axis_index("s")
    wid = s * NC + c

    srow0 = pl.multiple_of(s * RPS, 8)
    row0 = pl.multiple_of(c * NP + s * RPS, 8)
    pltpu.sync_copy(zc_hbm, cnt_sh.at[pl.ds(srow0, RPS), :])
    pltpu.sync_copy(ones_hbm, ones_v)
    plsc.subcore_barrier()

    base = wid * EPW

    def chunk(j, carry):
      off = base + j * K
      pltpu.sync_copy(dst_hbm.at[pl.ds(off, K)], dst_idx)
      pltpu.sync_copy(ones_v, cnt_sh.at[dst_idx], add=True)
      return carry

    lax.fori_loop(0, NCHUNK, chunk, 0)
    plsc.subcore_barrier()

    pltpu.sync_copy(cnt_sh.at[pl.ds(srow0, RPS), :],
                    cnt_out.at[pl.ds(row0, RPS), :])

  return body(dst, zc, ones_h)


def _tc_sage(x, sum_parts, cnt_parts, w1t, w2t, b2):
  R = 1000

  def body(x_ref, sp_ref, cp_ref, w1_ref, w2_ref, b_ref, o_ref):
    xb = x_ref[...]
    summed = sp_ref[0] + sp_ref[1]
    cnt = cp_ref[0, :, 0:1] + cp_ref[1, :, 0:1]
    mean = summed / jnp.maximum(cnt, 1.0)
    agg = jnp.where(cnt > 0.0, mean, xb)
    h = (jnp.dot(xb, w1_ref[...], preferred_element_type=jnp.float32)
         + jnp.dot(agg, w2_ref[...], preferred_element_type=jnp.float32)
         + b_ref[...])
    o_ref[...] = jnp.maximum(h, 0.0)

  return pl.pallas_call(
      body,
      grid=(N // R,),
      in_specs=[
          pl.BlockSpec((R, D), lambda i: (i, 0)),
          pl.BlockSpec((NC, R, D), lambda i: (0, i, 0)),
          pl.BlockSpec((NC, R, D), lambda i: (0, i, 0)),
          pl.BlockSpec((D, D), lambda i: (0, 0)),
          pl.BlockSpec((D, D), lambda i: (0, 0)),
          pl.BlockSpec((1, D), lambda i: (0, 0)),
      ],
      out_specs=pl.BlockSpec((R, D), lambda i: (i, 0)),
      out_shape=jax.ShapeDtypeStruct((N, D), jnp.float32),
  )(x, sum_parts, cnt_parts, w1t, w2t, b2)


def kernel(x, edge_index, W, b):
  src = edge_index[0].astype(jnp.int32)
  dst = edge_index[1].astype(jnp.int32)
  z2d = jnp.zeros((RPS, D), jnp.float32)
  zc = jnp.zeros((RPS, CW), jnp.float32)
  ones_h = jnp.ones((K, CW), jnp.float32)
  sum_flat = _sc_segment_sum(src, dst, x, z2d)
  zc2 = z2d + sum_flat[:RPS, :] * 0.0
  ones2d = jnp.ones((K, D), jnp.float32)
  cnt_flat = _sc_degree(dst, zc2, ones2d)
  sum_parts = sum_flat.reshape(NC, NP, D)
  cnt_parts = cnt_flat.reshape(NC, NP, D)
  w1t = W[:, :D].T
  w2t = W[:, D:].T
  b2 = b[None, :]
  return _tc_sage(x, sum_parts, cnt_parts, w1t, w2t, b2)


# speedup vs baseline: 9.5489x; 2.0145x over previous
import functools

import jax
import jax.numpy as jnp
from jax import lax
from jax.experimental import pallas as pl
from jax.experimental.pallas import tpu as pltpu
from jax.experimental.pallas import tpu_sc as plsc

N = 10000
E = 320000
D = 128
CW = 16

NC = 2
NS = 16
NW = NC * NS
EPW = E // NW
K = 80
NCHUNK = EPW // K
NP = 10240
RPS = NP // NS

_MESH = dict(core_axis_name="c", subcore_axis_name="s", num_cores=NC,
             num_subcores=NS)


def _sc_segment_sum(src, dst, x, z2d):
  @functools.partial(
      pl.kernel,
      mesh=plsc.VectorSubcoreMesh(**_MESH),
      out_type=jax.ShapeDtypeStruct((NC * NP, D), jnp.float32),
      scratch_types=[
          pltpu.VMEM_SHARED((NP, D), jnp.float32),
          pltpu.VMEM((EPW,), jnp.int32),
          pltpu.VMEM((K,), jnp.int32),
          pltpu.VMEM((K,), jnp.int32),
          pltpu.VMEM((K, D), jnp.float32),
          pltpu.VMEM((K, D), jnp.float32),
          pltpu.SemaphoreType.DMA,
          pltpu.SemaphoreType.DMA,
          pltpu.SemaphoreType.DMA,
          pltpu.SemaphoreType.DMA,
      ],
  )
  def body(src_hbm, dst_hbm, x_hbm, z2d_hbm, sum_out, acc_sh,
           src_all, dstb0, dstb1, rows0, rows1, sg0, sg1, sd0, sd1):
    c = lax.axis_index("c")
    s = lax.axis_index("s")
    wid = s * NC + c

    srow0 = pl.multiple_of(s * RPS, 8)
    row0 = pl.multiple_of(c * NP + s * RPS, 8)
    base = pl.multiple_of(wid * EPW, 8)
    pltpu.sync_copy(z2d_hbm, acc_sh.at[pl.ds(srow0, RPS), :])
    pltpu.sync_copy(src_hbm.at[pl.ds(base, EPW)], src_all)
    plsc.subcore_barrier()

    def gidx(j):
      return src_all.at[pl.ds(j * K, K)]

    def dslice(j):
      return dst_hbm.at[pl.ds(base + j * K, K)]

    pltpu.async_copy(dslice(0), dstb0, sd0)
    pltpu.async_copy(x_hbm.at[gidx(0)], rows0, sg0)

    def pair(p, carry):
      j0 = 2 * p
      dd1 = pltpu.async_copy(dslice(j0 + 1), dstb1, sd1)
      dg1 = pltpu.async_copy(x_hbm.at[gidx(j0 + 1)], rows1, sg1)
      pltpu.make_async_copy(dslice(j0), dstb0, sd0).wait()
      pltpu.make_async_copy(x_hbm.at[gidx(j0)], rows0, sg0).wait()
      pltpu.sync_copy(rows0, acc_sh.at[dstb0], add=True)
      pltpu.async_copy(dslice(j0 + 2), dstb0, sd0)
      pltpu.async_copy(x_hbm.at[gidx(j0 + 2)], rows0, sg0)
      dd1.wait()
      dg1.wait()
      pltpu.sync_copy(rows1, acc_sh.at[dstb1], add=True)
      return carry

    lax.fori_loop(0, (NCHUNK - 1) // 2, pair, 0)
    pltpu.make_async_copy(dslice(NCHUNK - 1), dstb0, sd0).wait()
    pltpu.make_async_copy(x_hbm.at[gidx(NCHUNK - 1)], rows0, sg0).wait()
    pltpu.sync_copy(rows0, acc_sh.at[dstb0], add=True)
    plsc.subcore_barrier()

    pltpu.sync_copy(acc_sh.at[pl.ds(srow0, RPS), :],
                    sum_out.at[pl.ds(row0, RPS), :])

  return body(src, dst, x, z2d)


def _sc_degree(dst, zc, ones_h):
  @functools.partial(
      pl.kernel,
      mesh=plsc.VectorSubcoreMesh(**_MESH),
      out_type=jax.ShapeDtypeStruct((NC * NP, D), jnp.float32),
      scratch_types=[
          pltpu.VMEM_SHARED((NP, D), jnp.float32),
          pltpu.VMEM((K,), jnp.int32),
          pltpu.VMEM((K,), jnp.int32),
          pltpu.VMEM((K, D), jnp.float32),
          pltpu.SemaphoreType.DMA,
          pltpu.SemaphoreType.DMA,
      ],
  )
  def body(dst_hbm, zc_hbm, ones_hbm, cnt_out, cnt_sh, dstb0, dstb1,
           ones_v, sd0, sd1):
    c = lax.axis_index("c")
    s = lax.axis_index("s")
    wid = s * NC + c

    srow0 = pl.multiple_of(s * RPS, 8)
    row0 = pl.multiple_of(c * NP + s * RPS, 8)
    base = pl.multiple_of(wid * EPW, 8)
    pltpu.sync_copy(zc_hbm, cnt_sh.at[pl.ds(srow0, RPS), :])
    pltpu.sync_copy(ones_hbm, ones_v)
    plsc.subcore_barrier()

    def dslice(j):
      return dst_hbm.at[pl.ds(base + j * K, K)]

    pltpu.async_copy(dslice(0), dstb0, sd0)

    def pair(p, carry):
      j0 = 2 * p
      dd1 = pltpu.async_copy(dslice(j0 + 1), dstb1, sd1)
      pltpu.make_async_copy(dslice(j0), dstb0, sd0).wait()
      pltpu.sync_copy(ones_v, cnt_sh.at[dstb0], add=True)
      pltpu.async_copy(dslice(j0 + 2), dstb0, sd0)
      dd1.wait()
      pltpu.sync_copy(ones_v, cnt_sh.at[dstb1], add=True)
      return carry

    lax.fori_loop(0, (NCHUNK - 1) // 2, pair, 0)
    pltpu.make_async_copy(dslice(NCHUNK - 1), dstb0, sd0).wait()
    pltpu.sync_copy(ones_v, cnt_sh.at[dstb0], add=True)
    plsc.subcore_barrier()

    pltpu.sync_copy(cnt_sh.at[pl.ds(srow0, RPS), :],
                    cnt_out.at[pl.ds(row0, RPS), :])

  return body(dst, zc, ones_h)


def _tc_sage(x, sum_parts, cnt_parts, w1t, w2t, b2):
  R = 1000

  def body(x_ref, sp_ref, cp_ref, w1_ref, w2_ref, b_ref, o_ref):
    xb = x_ref[...]
    summed = sp_ref[0] + sp_ref[1]
    cnt = cp_ref[0, :, 0:1] + cp_ref[1, :, 0:1]
    mean = summed / jnp.maximum(cnt, 1.0)
    agg = jnp.where(cnt > 0.0, mean, xb)
    h = (jnp.dot(xb, w1_ref[...], preferred_element_type=jnp.float32)
         + jnp.dot(agg, w2_ref[...], preferred_element_type=jnp.float32)
         + b_ref[...])
    o_ref[...] = jnp.maximum(h, 0.0)

  return pl.pallas_call(
      body,
      grid=(N // R,),
      in_specs=[
          pl.BlockSpec((R, D), lambda i: (i, 0)),
          pl.BlockSpec((NC, R, D), lambda i: (0, i, 0)),
          pl.BlockSpec((NC, R, D), lambda i: (0, i, 0)),
          pl.BlockSpec((D, D), lambda i: (0, 0)),
          pl.BlockSpec((D, D), lambda i: (0, 0)),
          pl.BlockSpec((1, D), lambda i: (0, 0)),
      ],
      out_specs=pl.BlockSpec((R, D), lambda i: (i, 0)),
      out_shape=jax.ShapeDtypeStruct((N, D), jnp.float32),
  )(x, sum_parts, cnt_parts, w1t, w2t, b2)


def kernel(x, edge_index, W, b):
  src = edge_index[0].astype(jnp.int32)
  dst = edge_index[1].astype(jnp.int32)
  z2d = jnp.zeros((RPS, D), jnp.float32)
  sum_flat = _sc_segment_sum(src, dst, x, z2d)
  zc2 = z2d + sum_flat[:RPS, :] * 0.0
  ones2d = jnp.ones((K, D), jnp.float32)
  cnt_flat = _sc_degree(dst, zc2, ones2d)
  sum_parts = sum_flat.reshape(NC, NP, D)
  cnt_parts = cnt_flat.reshape(NC, NP, D)
  w1t = W[:, :D].T
  w2t = W[:, D:].T
  b2 = b[None, :]
  return _tc_sage(x, sum_parts, cnt_parts, w1t, w2t, b2)
